# baseline (speedup 1.0000x reference)
import functools

import jax
import jax.numpy as jnp
from jax import lax
from jax.experimental import pallas as pl
from jax.experimental.pallas import tpu as pltpu
from jax.experimental.pallas import tpu_sc as plsc

N_NODES = 10000
N_EDGES = 320000
D = 128
DW = D // 2

NW = 32
R = 313
N_PAD = NW * R
CHUNK = 16000
N_CHUNKS = N_EDGES // CHUNK
VPC = CHUNK // 16
UN = 4
CAP = 12000
MAXC = CAP - 64
TRASH = CAP + 64
HBUF = CAP + 96
G = 64


def _wid():
    return lax.axis_index("s") * 2 + lax.axis_index("c")


def _sc_mesh():
    return plsc.VectorSubcoreMesh(core_axis_name="c", subcore_axis_name="s")


_SC_PARAMS = pltpu.CompilerParams(needs_layout_passes=False,
                                  use_tc_tiling_on_sc=False)


def _prep_body(src_hbm, dst_hbm, lists_hbm, counts_hbm,
               srcA, dstA, srcB, dstB, hits, semA, semB):
    w = _wid()
    lo = w * R

    def fire(c, sbuf, dbuf, sem):
        c = jnp.minimum(c, N_CHUNKS - 1)
        pltpu.async_copy(src_hbm.at[pl.ds(c * CHUNK, CHUNK)], sbuf, sem)
        pltpu.async_copy(dst_hbm.at[pl.ds(c * CHUNK, CHUNK)], dbuf, sem)

    def drain(sbuf, dbuf, sem):
        pltpu.make_async_copy(src_hbm.at[pl.ds(0, CHUNK)], sbuf, sem).wait()
        pltpu.make_async_copy(dst_hbm.at[pl.ds(0, CHUNK)], dbuf, sem).wait()

    def scan_chunk(srcc, dstc, off):
        def scan_body(v, off):
            for u in range(UN):
                b = (v * UN + u) * 16
                d = dstc[pl.ds(b, 16)]
                s = srcc[pl.ds(b, 16)]
                dl = d - lo
                m = (dl >= 0) & (dl < R)
                c16 = plsc.cumsum(jnp.where(m, 1, 0).astype(jnp.int32))
                pos = jnp.where(m, c16 + (off - 1), TRASH)
                pos = jnp.minimum(pos, TRASH)
                packed = lax.shift_left(s, 9) | dl
                plsc.store_scatter(hits, [pos], packed)
                off = off + plsc.all_reduce_population_count(m)[0]
            return off

        return lax.fori_loop(0, VPC // UN, scan_body, off)

    fire(0, srcA, dstA, semA)

    def pair_body(i, off):
        c0 = 2 * i
        fire(c0 + 1, srcB, dstB, semB)
        drain(srcA, dstA, semA)
        off = scan_chunk(srcA, dstA, off)
        fire(c0 + 2, srcA, dstA, semA)
        drain(srcB, dstB, semB)
        return scan_chunk(srcB, dstB, off)

    off = lax.fori_loop(0, N_CHUNKS // 2, pair_body, jnp.int32(0))
    drain(srcA, dstA, semA)

    off = jnp.minimum(off, MAXC)
    pad = jnp.full((16,), R, jnp.int32)
    for u in range(4):
        hits[pl.ds(off + u * 16, 16)] = pad
    pltpu.sync_copy(hits.at[pl.ds(0, CAP)], lists_hbm.at[pl.ds(w * CAP, CAP)])
    hits[pl.ds(HBUF - 16, 16)] = jnp.full((16,), off, jnp.int32)
    pltpu.sync_copy(hits.at[pl.ds(HBUF - 16, 16)],
                    counts_hbm.at[pl.ds(w * 16, 16)])


def _prep(src, dst):
    k = pl.kernel(
        _prep_body,
        out_type=(
            jax.ShapeDtypeStruct((NW * CAP,), jnp.int32),
            jax.ShapeDtypeStruct((NW * 16,), jnp.int32),
        ),
        mesh=_sc_mesh(),
        compiler_params=_SC_PARAMS,
        scratch_types=[
            pltpu.VMEM((CHUNK,), jnp.int32),
            pltpu.VMEM((CHUNK,), jnp.int32),
            pltpu.VMEM((CHUNK,), jnp.int32),
            pltpu.VMEM((CHUNK,), jnp.int32),
            pltpu.VMEM((HBUF,), jnp.int32),
            pltpu.SemaphoreType.DMA,
            pltpu.SemaphoreType.DMA,
        ],
    )
    return k(src, dst)


def _segmax_body(hp_hbm, lists_hbm, counts_hbm, out_hbm,
                 listv, srcb, dlb, acc, rows0, rows1, rows2, rows3, cntv,
                 sem0, sem1, sem2, sem3):
    w = _wid()
    lo = w * R

    pltpu.sync_copy(counts_hbm.at[pl.ds(w * 16, 16)], cntv)
    cnt = cntv[pl.ds(0, 16)][0]
    ngroups = (cnt + (G - 1)) // G
    pltpu.sync_copy(lists_hbm.at[pl.ds(w * CAP, CAP)], listv)

    def unpack_body(v, _):
        p = listv[pl.ds(v * 16, 16)]
        srcb[pl.ds(v * 16, 16)] = lax.shift_right_logical(p, 9)
        dlb[pl.ds(v * 16, 16)] = lax.shift_left(p & 511, 6)
        return 0

    lax.fori_loop(0, ngroups * (G // 16), unpack_body, 0)

    zeros16 = jnp.zeros((16,), jnp.int32)

    def zero_body(i, _):
        acc[pl.ds(i * 16, 16)] = zeros16
        return 0

    lax.fori_loop(0, (R + 1) * DW // 16, zero_body, 0)

    bufs = ((rows0, sem0), (rows1, sem1), (rows2, sem2), (rows3, sem3))

    def fire(g, k):
        g = jnp.minimum(g, ngroups - 1)
        rbuf, sem = bufs[k]
        pltpu.async_copy(hp_hbm.at[srcb.at[pl.ds(g * G, G)]], rbuf, sem)

    def accum(g, k):
        rbuf = bufs[k][0]
        for q in range(G // 16):
            dls = dlb[pl.ds(g * G + q * 16, 16)]
            for e in range(16):
                base = dls[e]
                for f in range(DW // 16):
                    sl = pl.ds(base + f * 16, 16)
                    a = plsc.bitcast(acc[sl], jnp.bfloat16)
                    r = plsc.bitcast(rbuf[q * 16 + e, pl.ds(f * 16, 16)],
                                     jnp.bfloat16)
                    acc[sl] = plsc.bitcast(jnp.maximum(a, r), jnp.int32)

    def wait(k):
        rbuf, sem = bufs[k]
        pltpu.make_async_copy(
            hp_hbm.at[srcb.at[pl.ds(0, G)]], rbuf, sem).wait()

    for k in range(3):
        fire(jnp.int32(k), k)

    def quad_body(i, _):
        g0 = 4 * i
        for k in range(4):
            fire(g0 + k + 3, (k + 3) % 4)
            wait(k)
            accum(jnp.minimum(g0 + k, ngroups - 1), k)
        return 0

    lax.fori_loop(0, (ngroups + 3) // 4, quad_body, 0)
    for k in range(3):
        wait(k)

    pltpu.sync_copy(acc.at[pl.ds(0, R * DW)],
                    out_hbm.at[pl.ds(lo * DW, R * DW)])


def _segmax(hp, lists, counts):
    k = pl.kernel(
        _segmax_body,
        out_type=jax.ShapeDtypeStruct((N_PAD * DW,), jnp.int32),
        mesh=_sc_mesh(),
        compiler_params=_SC_PARAMS,
        scratch_types=[
            pltpu.VMEM((CAP,), jnp.int32),
            pltpu.VMEM((CAP,), jnp.int32),
            pltpu.VMEM((CAP,), jnp.int32),
            pltpu.VMEM(((R + 1) * DW,), jnp.int32),
            pltpu.VMEM((G, DW), jnp.int32),
            pltpu.VMEM((G, DW), jnp.int32),
            pltpu.VMEM((G, DW), jnp.int32),
            pltpu.VMEM((G, DW), jnp.int32),
            pltpu.VMEM((16,), jnp.int32),
            pltpu.SemaphoreType.DMA,
            pltpu.SemaphoreType.DMA,
            pltpu.SemaphoreType.DMA,
            pltpu.SemaphoreType.DMA,
        ],
    )
    out = k(hp, lists, counts).reshape(N_PAD, DW)[:N_NODES]
    return lax.bitcast_convert_type(out, jnp.bfloat16).reshape(N_NODES, D)


def _mm_pre_body(h_ref, wp_ref, bp_ref, ws_ref, hp_ref, self_ref):
    h = h_ref[...]
    hp_ref[...] = jnp.maximum(
        jnp.dot(h, wp_ref[...], preferred_element_type=jnp.float32)
        + bp_ref[...], 0.0).astype(jnp.bfloat16)
    self_ref[...] = jnp.dot(h, ws_ref[...], preferred_element_type=jnp.float32)


def _mm_pre(h, Wp, bp, Ws):
    blk = N_NODES // 10
    return pl.pallas_call(
        _mm_pre_body,
        grid=(10,),
        in_specs=[
            pl.BlockSpec((blk, D), lambda i: (i, 0)),
            pl.BlockSpec((D, D), lambda i: (0, 0)),
            pl.BlockSpec((1, D), lambda i: (0, 0)),
            pl.BlockSpec((D, D), lambda i: (0, 0)),
        ],
        out_specs=[
            pl.BlockSpec((blk, D), lambda i: (i, 0)),
            pl.BlockSpec((blk, D), lambda i: (i, 0)),
        ],
        out_shape=[
            jax.ShapeDtypeStruct((N_NODES, D), jnp.bfloat16),
            jax.ShapeDtypeStruct((N_NODES, D), jnp.float32),
        ],
    )(h, Wp, bp.reshape(1, D), Ws)


def _mm_post_body(self_ref, neigh_ref, wn_ref, b_ref, out_ref, *, norm):
    out = self_ref[...] + jnp.dot(
        neigh_ref[...].astype(jnp.float32), wn_ref[...],
        preferred_element_type=jnp.float32,
    ) + b_ref[...]
    if norm:
        out = jnp.maximum(out, 0.0)
        n = jnp.sqrt(jnp.sum(out * out, axis=1, keepdims=True))
        out = out / jnp.maximum(n, 1e-12)
    out_ref[...] = out


def _mm_post(selfo, neigh, Wn, b, norm):
    blk = N_NODES // 10
    return pl.pallas_call(
        functools.partial(_mm_post_body, norm=norm),
        grid=(10,),
        in_specs=[
            pl.BlockSpec((blk, D), lambda i: (i, 0)),
            pl.BlockSpec((blk, D), lambda i: (i, 0)),
            pl.BlockSpec((D, D), lambda i: (0, 0)),
            pl.BlockSpec((1, D), lambda i: (0, 0)),
        ],
        out_specs=pl.BlockSpec((blk, D), lambda i: (i, 0)),
        out_shape=jax.ShapeDtypeStruct((N_NODES, D), jnp.float32),
    )(selfo, neigh, Wn, b.reshape(1, D))


def kernel(inputs, edge_index, Wp0, bp0, Wn0, Ws0, b0, Wp1, bp1, Wn1, Ws1, b1):
    src = edge_index[0]
    dst = edge_index[1]
    lists, counts = _prep(src, dst)
    h = inputs
    for Wp, bp, Wn, Ws, b, norm in (
        (Wp0, bp0, Wn0, Ws0, b0, True),
        (Wp1, bp1, Wn1, Ws1, b1, False),
    ):
        hp, selfo = _mm_pre(h, Wp, bp, Ws)
        hp_w = lax.bitcast_convert_type(hp.reshape(N_NODES, DW, 2), jnp.int32)
        neigh = _segmax(hp_w, lists, counts)
        h = _mm_post(selfo, neigh, Wn, b, norm)
    return h


# speedup vs baseline: 1.1098x; 1.1098x over previous
import functools

import jax
import jax.numpy as jnp
from jax import lax
from jax.experimental import pallas as pl
from jax.experimental.pallas import tpu as pltpu
from jax.experimental.pallas import tpu_sc as plsc

N_NODES = 10000
N_EDGES = 320000
D = 128
DW = D // 2

NW = 32
R = 313
N_PAD = NW * R
CHUNK = 16000
N_CHUNKS = N_EDGES // CHUNK
VPC = CHUNK // 16
UN = 4
CAP = 12000
MAXC = CAP - 64
TRASH = CAP + 64
HBUF = CAP + 96
G = 64


def _wid():
    return lax.axis_index("s") * 2 + lax.axis_index("c")


def _sc_mesh():
    return plsc.VectorSubcoreMesh(core_axis_name="c", subcore_axis_name="s")


_SC_PARAMS = pltpu.CompilerParams(needs_layout_passes=False,
                                  use_tc_tiling_on_sc=False)


def _prep_body(src_hbm, dst_hbm, lists_hbm, counts_hbm,
               srcA, dstA, srcB, dstB, hits, semA, semB):
    w = _wid()
    lo = w * R

    def fire(c, sbuf, dbuf, sem):
        c = jnp.minimum(c, N_CHUNKS - 1)
        pltpu.async_copy(src_hbm.at[pl.ds(c * CHUNK, CHUNK)], sbuf, sem)
        pltpu.async_copy(dst_hbm.at[pl.ds(c * CHUNK, CHUNK)], dbuf, sem)

    def drain(sbuf, dbuf, sem):
        pltpu.make_async_copy(src_hbm.at[pl.ds(0, CHUNK)], sbuf, sem).wait()
        pltpu.make_async_copy(dst_hbm.at[pl.ds(0, CHUNK)], dbuf, sem).wait()

    def scan_chunk(srcc, dstc, off):
        def scan_body(v, off):
            for u in range(UN):
                b = (v * UN + u) * 16
                d = dstc[pl.ds(b, 16)]
                s = srcc[pl.ds(b, 16)]
                dl = d - lo
                m = (dl >= 0) & (dl < R)
                c16 = plsc.cumsum(jnp.where(m, 1, 0).astype(jnp.int32))
                pos = jnp.where(m, c16 + (off - 1), TRASH)
                pos = jnp.minimum(pos, TRASH)
                packed = lax.shift_left(s, 9) | dl
                plsc.store_scatter(hits, [pos], packed)
                off = off + plsc.all_reduce_population_count(m)[0]
            return off

        return lax.fori_loop(0, VPC // UN, scan_body, off)

    fire(0, srcA, dstA, semA)

    def pair_body(i, off):
        c0 = 2 * i
        fire(c0 + 1, srcB, dstB, semB)
        drain(srcA, dstA, semA)
        off = scan_chunk(srcA, dstA, off)
        fire(c0 + 2, srcA, dstA, semA)
        drain(srcB, dstB, semB)
        return scan_chunk(srcB, dstB, off)

    off = lax.fori_loop(0, N_CHUNKS // 2, pair_body, jnp.int32(0))
    drain(srcA, dstA, semA)

    off = jnp.minimum(off, MAXC)
    pad = jnp.full((16,), R, jnp.int32)
    for u in range(4):
        hits[pl.ds(off + u * 16, 16)] = pad
    pltpu.sync_copy(hits.at[pl.ds(0, CAP)], lists_hbm.at[pl.ds(w * CAP, CAP)])
    hits[pl.ds(HBUF - 16, 16)] = jnp.full((16,), off, jnp.int32)
    pltpu.sync_copy(hits.at[pl.ds(HBUF - 16, 16)],
                    counts_hbm.at[pl.ds(w * 16, 16)])


def _prep(src, dst):
    k = pl.kernel(
        _prep_body,
        out_type=(
            jax.ShapeDtypeStruct((NW * CAP,), jnp.int32),
            jax.ShapeDtypeStruct((NW * 16,), jnp.int32),
        ),
        mesh=_sc_mesh(),
        compiler_params=_SC_PARAMS,
        scratch_types=[
            pltpu.VMEM((CHUNK,), jnp.int32),
            pltpu.VMEM((CHUNK,), jnp.int32),
            pltpu.VMEM((CHUNK,), jnp.int32),
            pltpu.VMEM((CHUNK,), jnp.int32),
            pltpu.VMEM((HBUF,), jnp.int32),
            pltpu.SemaphoreType.DMA,
            pltpu.SemaphoreType.DMA,
        ],
    )
    return k(src, dst)


def _segmax_body(hp_hbm, lists_hbm, counts_hbm, out_hbm,
                 listv, srcb, dlb, acc, rows0, rows1, cntv, sem0, sem1):
    w = _wid()
    lo = w * R

    pltpu.sync_copy(counts_hbm.at[pl.ds(w * 16, 16)], cntv)
    cnt = cntv[pl.ds(0, 16)][0]
    ngroups = (cnt + (G - 1)) // G
    pltpu.sync_copy(lists_hbm.at[pl.ds(w * CAP, CAP)], listv)

    def unpack_body(v, _):
        p = listv[pl.ds(v * 16, 16)]
        srcb[pl.ds(v * 16, 16)] = lax.shift_right_logical(p, 9)
        dlb[pl.ds(v * 16, 16)] = lax.shift_left(p & 511, 6)
        return 0

    lax.fori_loop(0, ngroups * (G // 16), unpack_body, 0)

    zeros16 = jnp.zeros((16,), jnp.int32)

    def zero_body(i, _):
        acc[pl.ds(i * 16, 16)] = zeros16
        return 0

    lax.fori_loop(0, (R + 1) * DW // 16, zero_body, 0)

    bufs = ((rows0, sem0), (rows1, sem1))

    def fire(g, k):
        g = jnp.minimum(g, ngroups - 1)
        rbuf, sem = bufs[k]
        for j in range(G // 16):
            pltpu.async_copy(hp_hbm.at[srcb.at[pl.ds(g * G + j * 16, 16)]],
                             rbuf.at[pl.ds(j * 16, 16)], sem)

    def accum(g, k):
        rbuf = bufs[k][0]
        for q in range(G // 16):
            dls = dlb[pl.ds(g * G + q * 16, 16)]
            for e in range(16):
                base = dls[e]
                for f in range(DW // 16):
                    sl = pl.ds(base + f * 16, 16)
                    a = plsc.bitcast(acc[sl], jnp.bfloat16)
                    r = plsc.bitcast(rbuf[q * 16 + e, pl.ds(f * 16, 16)],
                                     jnp.bfloat16)
                    acc[sl] = plsc.bitcast(jnp.maximum(a, r), jnp.int32)

    def wait(k):
        rbuf, sem = bufs[k]
        for j in range(G // 16):
            pltpu.make_async_copy(
                hp_hbm.at[srcb.at[pl.ds(0, 16)]],
                rbuf.at[pl.ds(j * 16, 16)], sem).wait()

    fire(jnp.int32(0), 0)

    def pair_body(i, _):
        g0 = 2 * i
        fire(g0 + 1, 1)
        wait(0)
        accum(g0, 0)
        fire(g0 + 2, 0)
        wait(1)
        accum(jnp.minimum(g0 + 1, ngroups - 1), 1)
        return 0

    lax.fori_loop(0, (ngroups + 1) // 2, pair_body, 0)
    wait(0)

    pltpu.sync_copy(acc.at[pl.ds(0, R * DW)],
                    out_hbm.at[pl.ds(lo * DW, R * DW)])


def _segmax(hp, lists, counts):
    k = pl.kernel(
        _segmax_body,
        out_type=jax.ShapeDtypeStruct((N_PAD * DW,), jnp.int32),
        mesh=_sc_mesh(),
        compiler_params=_SC_PARAMS,
        scratch_types=[
            pltpu.VMEM((CAP,), jnp.int32),
            pltpu.VMEM((CAP,), jnp.int32),
            pltpu.VMEM((CAP,), jnp.int32),
            pltpu.VMEM(((R + 1) * DW,), jnp.int32),
            pltpu.VMEM((G, DW), jnp.int32),
            pltpu.VMEM((G, DW), jnp.int32),
            pltpu.VMEM((16,), jnp.int32),
            pltpu.SemaphoreType.DMA,
            pltpu.SemaphoreType.DMA,
        ],
    )
    out = k(hp, lists, counts).reshape(N_PAD, DW)[:N_NODES]
    return lax.bitcast_convert_type(out, jnp.bfloat16).reshape(N_NODES, D)


def _mm_pre_body(h_ref, wp_ref, bp_ref, ws_ref, hp_ref, self_ref):
    h = h_ref[...]
    hp_ref[...] = jnp.maximum(
        jnp.dot(h, wp_ref[...], preferred_element_type=jnp.float32)
        + bp_ref[...], 0.0).astype(jnp.bfloat16)
    self_ref[...] = jnp.dot(h, ws_ref[...], preferred_element_type=jnp.float32)


def _mm_pre(h, Wp, bp, Ws):
    blk = N_NODES // 10
    return pl.pallas_call(
        _mm_pre_body,
        grid=(10,),
        in_specs=[
            pl.BlockSpec((blk, D), lambda i: (i, 0)),
            pl.BlockSpec((D, D), lambda i: (0, 0)),
            pl.BlockSpec((1, D), lambda i: (0, 0)),
            pl.BlockSpec((D, D), lambda i: (0, 0)),
        ],
        out_specs=[
            pl.BlockSpec((blk, D), lambda i: (i, 0)),
            pl.BlockSpec((blk, D), lambda i: (i, 0)),
        ],
        out_shape=[
            jax.ShapeDtypeStruct((N_NODES, D), jnp.bfloat16),
            jax.ShapeDtypeStruct((N_NODES, D), jnp.float32),
        ],
    )(h, Wp, bp.reshape(1, D), Ws)


def _mm_post_body(self_ref, neigh_ref, wn_ref, b_ref, out_ref, *, norm):
    out = self_ref[...] + jnp.dot(
        neigh_ref[...].astype(jnp.float32), wn_ref[...],
        preferred_element_type=jnp.float32,
    ) + b_ref[...]
    if norm:
        out = jnp.maximum(out, 0.0)
        n = jnp.sqrt(jnp.sum(out * out, axis=1, keepdims=True))
        out = out / jnp.maximum(n, 1e-12)
    out_ref[...] = out


def _mm_post(selfo, neigh, Wn, b, norm):
    blk = N_NODES // 10
    return pl.pallas_call(
        functools.partial(_mm_post_body, norm=norm),
        grid=(10,),
        in_specs=[
            pl.BlockSpec((blk, D), lambda i: (i, 0)),
            pl.BlockSpec((blk, D), lambda i: (i, 0)),
            pl.BlockSpec((D, D), lambda i: (0, 0)),
            pl.BlockSpec((1, D), lambda i: (0, 0)),
        ],
        out_specs=pl.BlockSpec((blk, D), lambda i: (i, 0)),
        out_shape=jax.ShapeDtypeStruct((N_NODES, D), jnp.float32),
    )(selfo, neigh, Wn, b.reshape(1, D))


def kernel(inputs, edge_index, Wp0, bp0, Wn0, Ws0, b0, Wp1, bp1, Wn1, Ws1, b1):
    src = edge_index[0]
    dst = edge_index[1]
    lists, counts = _prep(src, dst)
    h = inputs
    for Wp, bp, Wn, Ws, b, norm in (
        (Wp0, bp0, Wn0, Ws0, b0, True),
        (Wp1, bp1, Wn1, Ws1, b1, False),
    ):
        hp, selfo = _mm_pre(h, Wp, bp, Ws)
        hp_w = lax.bitcast_convert_type(hp.reshape(N_NODES, DW, 2), jnp.int32)
        neigh = _segmax(hp_w, lists, counts)
        h = _mm_post(selfo, neigh, Wn, b, norm)
    return h


# speedup vs baseline: 1.1168x; 1.0063x over previous
import functools

import jax
import jax.numpy as jnp
from jax import lax
from jax.experimental import pallas as pl
from jax.experimental.pallas import tpu as pltpu
from jax.experimental.pallas import tpu_sc as plsc

N_NODES = 10000
N_EDGES = 320000
D = 128
DW = D // 2

NW = 32
R = 313
N_PAD = NW * R
CHUNK = 16000
N_CHUNKS = N_EDGES // CHUNK
VPC = CHUNK // 16
UN = 4
CAP = 12000
MAXC = CAP - 64
TRASH = CAP + 64
HBUF = CAP + 96
G = 64


def _wid():
    return lax.axis_index("s") * 2 + lax.axis_index("c")


def _sc_mesh():
    return plsc.VectorSubcoreMesh(core_axis_name="c", subcore_axis_name="s")


_SC_PARAMS = pltpu.CompilerParams(needs_layout_passes=False,
                                  use_tc_tiling_on_sc=False)


def _prep_body(src_hbm, dst_hbm, lists_hbm, counts_hbm,
               srcA, dstA, srcB, dstB, hits, semA, semB):
    w = _wid()
    lo = w * R

    def fire(c, sbuf, dbuf, sem):
        c = jnp.minimum(c, N_CHUNKS - 1)
        pltpu.async_copy(src_hbm.at[pl.ds(c * CHUNK, CHUNK)], sbuf, sem)
        pltpu.async_copy(dst_hbm.at[pl.ds(c * CHUNK, CHUNK)], dbuf, sem)

    def drain(sbuf, dbuf, sem):
        pltpu.make_async_copy(src_hbm.at[pl.ds(0, CHUNK)], sbuf, sem).wait()
        pltpu.make_async_copy(dst_hbm.at[pl.ds(0, CHUNK)], dbuf, sem).wait()

    def scan_chunk(srcc, dstc, off):
        def scan_body(v, off):
            for u in range(UN):
                b = (v * UN + u) * 16
                d = dstc[pl.ds(b, 16)]
                s = srcc[pl.ds(b, 16)]
                dl = d - lo
                m = lax.bitcast_convert_type(dl, jnp.uint32) < jnp.uint32(R)
                c16 = plsc.cumsum(jnp.where(m, 1, 0).astype(jnp.int32))
                pos = jnp.where(m, c16 + (off - 1), TRASH)
                pos = jnp.minimum(pos, TRASH)
                packed = lax.shift_left(s, 9) | dl
                plsc.store_scatter(hits, [pos], packed)
                off = off + plsc.all_reduce_population_count(m)[0]
            return off

        return lax.fori_loop(0, VPC // UN, scan_body, off)

    fire(0, srcA, dstA, semA)

    def pair_body(i, off):
        c0 = 2 * i
        fire(c0 + 1, srcB, dstB, semB)
        drain(srcA, dstA, semA)
        off = scan_chunk(srcA, dstA, off)
        fire(c0 + 2, srcA, dstA, semA)
        drain(srcB, dstB, semB)
        return scan_chunk(srcB, dstB, off)

    off = lax.fori_loop(0, N_CHUNKS // 2, pair_body, jnp.int32(0))
    drain(srcA, dstA, semA)

    off = jnp.minimum(off, MAXC)
    pad = jnp.full((16,), R, jnp.int32)
    for u in range(4):
        hits[pl.ds(off + u * 16, 16)] = pad
    pltpu.sync_copy(hits.at[pl.ds(0, CAP)], lists_hbm.at[pl.ds(w * CAP, CAP)])
    hits[pl.ds(HBUF - 16, 16)] = jnp.full((16,), off, jnp.int32)
    pltpu.sync_copy(hits.at[pl.ds(HBUF - 16, 16)],
                    counts_hbm.at[pl.ds(w * 16, 16)])


def _prep(src, dst):
    k = pl.kernel(
        _prep_body,
        out_type=(
            jax.ShapeDtypeStruct((NW * CAP,), jnp.int32),
            jax.ShapeDtypeStruct((NW * 16,), jnp.int32),
        ),
        mesh=_sc_mesh(),
        compiler_params=_SC_PARAMS,
        scratch_types=[
            pltpu.VMEM((CHUNK,), jnp.int32),
            pltpu.VMEM((CHUNK,), jnp.int32),
            pltpu.VMEM((CHUNK,), jnp.int32),
            pltpu.VMEM((CHUNK,), jnp.int32),
            pltpu.VMEM((HBUF,), jnp.int32),
            pltpu.SemaphoreType.DMA,
            pltpu.SemaphoreType.DMA,
        ],
    )
    return k(src, dst)


def _segmax_body(hp_hbm, lists_hbm, counts_hbm, out_hbm,
                 listv, srcb, dlb, acc, rows0, rows1, cntv, sem0, sem1):
    w = _wid()
    lo = w * R

    pltpu.sync_copy(counts_hbm.at[pl.ds(w * 16, 16)], cntv)
    cnt = cntv[pl.ds(0, 16)][0]
    ngroups = (cnt + (G - 1)) // G
    pltpu.sync_copy(lists_hbm.at[pl.ds(w * CAP, CAP)], listv)

    def unpack_body(v, _):
        p = listv[pl.ds(v * 16, 16)]
        srcb[pl.ds(v * 16, 16)] = lax.shift_right_logical(p, 9)
        dlb[pl.ds(v * 16, 16)] = lax.shift_left(p & 511, 6)
        return 0

    lax.fori_loop(0, ngroups * (G // 16), unpack_body, 0)

    zeros16 = jnp.zeros((16,), jnp.int32)

    def zero_body(i, _):
        acc[pl.ds(i * 16, 16)] = zeros16
        return 0

    lax.fori_loop(0, (R + 1) * DW // 16, zero_body, 0)

    bufs = ((rows0, sem0), (rows1, sem1))

    def fire(g, k):
        g = jnp.minimum(g, ngroups - 1)
        rbuf, sem = bufs[k]
        pltpu.async_copy(hp_hbm.at[srcb.at[pl.ds(g * G, G)]], rbuf, sem)

    def accum(g, k):
        rbuf = bufs[k][0]
        for q in range(G // 16):
            dls = dlb[pl.ds(g * G + q * 16, 16)]
            for e in range(16):
                base = dls[e]
                for f in range(DW // 16):
                    sl = pl.ds(base + f * 16, 16)
                    a = plsc.bitcast(acc[sl], jnp.bfloat16)
                    r = plsc.bitcast(rbuf[q * 16 + e, pl.ds(f * 16, 16)],
                                     jnp.bfloat16)
                    acc[sl] = plsc.bitcast(jnp.maximum(a, r), jnp.int32)

    def wait(k):
        rbuf, sem = bufs[k]
        pltpu.make_async_copy(
            hp_hbm.at[srcb.at[pl.ds(0, G)]], rbuf, sem).wait()

    fire(jnp.int32(0), 0)

    def pair_body(i, _):
        g0 = 2 * i
        fire(g0 + 1, 1)
        wait(0)
        accum(g0, 0)
        fire(g0 + 2, 0)
        wait(1)
        accum(jnp.minimum(g0 + 1, ngroups - 1), 1)
        return 0

    lax.fori_loop(0, (ngroups + 1) // 2, pair_body, 0)
    wait(0)

    pltpu.sync_copy(acc.at[pl.ds(0, R * DW)],
                    out_hbm.at[pl.ds(lo * DW, R * DW)])


def _segmax(hp, lists, counts):
    k = pl.kernel(
        _segmax_body,
        out_type=jax.ShapeDtypeStruct((N_PAD * DW,), jnp.int32),
        mesh=_sc_mesh(),
        compiler_params=_SC_PARAMS,
        scratch_types=[
            pltpu.VMEM((CAP,), jnp.int32),
            pltpu.VMEM((CAP,), jnp.int32),
            pltpu.VMEM((CAP,), jnp.int32),
            pltpu.VMEM(((R + 1) * DW,), jnp.int32),
            pltpu.VMEM((G, DW), jnp.int32),
            pltpu.VMEM((G, DW), jnp.int32),
            pltpu.VMEM((16,), jnp.int32),
            pltpu.SemaphoreType.DMA,
            pltpu.SemaphoreType.DMA,
        ],
    )
    out = k(hp, lists, counts).reshape(N_PAD, DW)[:N_NODES]
    return lax.bitcast_convert_type(out, jnp.bfloat16).reshape(N_NODES, D)


def _mm_pre_body(h_ref, wp_ref, bp_ref, ws_ref, hp_ref, self_ref):
    h = h_ref[...]
    hp_ref[...] = jnp.maximum(
        jnp.dot(h, wp_ref[...], preferred_element_type=jnp.float32)
        + bp_ref[...], 0.0).astype(jnp.bfloat16)
    self_ref[...] = jnp.dot(h, ws_ref[...], preferred_element_type=jnp.float32)


def _mm_pre(h, Wp, bp, Ws):
    blk = N_NODES // 10
    return pl.pallas_call(
        _mm_pre_body,
        grid=(10,),
        in_specs=[
            pl.BlockSpec((blk, D), lambda i: (i, 0)),
            pl.BlockSpec((D, D), lambda i: (0, 0)),
            pl.BlockSpec((1, D), lambda i: (0, 0)),
            pl.BlockSpec((D, D), lambda i: (0, 0)),
        ],
        out_specs=[
            pl.BlockSpec((blk, D), lambda i: (i, 0)),
            pl.BlockSpec((blk, D), lambda i: (i, 0)),
        ],
        out_shape=[
            jax.ShapeDtypeStruct((N_NODES, D), jnp.bfloat16),
            jax.ShapeDtypeStruct((N_NODES, D), jnp.float32),
        ],
    )(h, Wp, bp.reshape(1, D), Ws)


def _mm_post_body(self_ref, neigh_ref, wn_ref, b_ref, out_ref, *, norm):
    out = self_ref[...] + jnp.dot(
        neigh_ref[...].astype(jnp.float32), wn_ref[...],
        preferred_element_type=jnp.float32,
    ) + b_ref[...]
    if norm:
        out = jnp.maximum(out, 0.0)
        n = jnp.sqrt(jnp.sum(out * out, axis=1, keepdims=True))
        out = out / jnp.maximum(n, 1e-12)
    out_ref[...] = out


def _mm_post(selfo, neigh, Wn, b, norm):
    blk = N_NODES // 10
    return pl.pallas_call(
        functools.partial(_mm_post_body, norm=norm),
        grid=(10,),
        in_specs=[
            pl.BlockSpec((blk, D), lambda i: (i, 0)),
            pl.BlockSpec((blk, D), lambda i: (i, 0)),
            pl.BlockSpec((D, D), lambda i: (0, 0)),
            pl.BlockSpec((1, D), lambda i: (0, 0)),
        ],
        out_specs=pl.BlockSpec((blk, D), lambda i: (i, 0)),
        out_shape=jax.ShapeDtypeStruct((N_NODES, D), jnp.float32),
    )(selfo, neigh, Wn, b.reshape(1, D))


def kernel(inputs, edge_index, Wp0, bp0, Wn0, Ws0, b0, Wp1, bp1, Wn1, Ws1, b1):
    src = edge_index[0]
    dst = edge_index[1]
    lists, counts = _prep(src, dst)
    h = inputs
    for Wp, bp, Wn, Ws, b, norm in (
        (Wp0, bp0, Wn0, Ws0, b0, True),
        (Wp1, bp1, Wn1, Ws1, b1, False),
    ):
        hp, selfo = _mm_pre(h, Wp, bp, Ws)
        hp_w = lax.bitcast_convert_type(hp.reshape(N_NODES, DW, 2), jnp.int32)
        neigh = _segmax(hp_w, lists, counts)
        h = _mm_post(selfo, neigh, Wn, b, norm)
    return h
